# Initial kernel scaffold; baseline (speedup 1.0000x reference)
import jax
import jax.numpy as jnp
from jax.experimental import pallas as pl


def kernel(x, edge_index, is_reversed, params):
    raise NotImplementedError("write your pallas kernel here")


# speedup vs baseline: 17.3333x; 17.3333x over previous
import functools

import jax
import jax.numpy as jnp
from jax import lax
from jax.experimental import pallas as pl
from jax.experimental.pallas import tpu as pltpu
from jax.experimental.pallas import tpu_sc as plsc

N = 10000
E = 160000
EPAD = 163840
D_FEAT = 256
HEADS = 4
CH = 128
NUM_CLASSES = 16
RB = 400
NTILE = 16
NSC = 2
NPAD = 10240
OPAD = NPAD + 128
ROWS_T = NPAD // NTILE
DROWS = NPAD // 128

_f32 = jnp.float32
_i32 = jnp.int32

_GDN = lax.GatherDimensionNumbers(offset_dims=(), collapsed_slice_dims=(0,),
                                  start_index_map=(0,))


def _vgather(vec, idx):
    return lax.gather(vec, idx[:, None], _GDN, slice_sizes=(1,),
                      mode=lax.GatherScatterMode.PROMISE_IN_BOUNDS)


def _mm_body(H, C, x_ref, w_ref, hf_ref, a_ref):
    m = jnp.dot(x_ref[...], w_ref[...], preferred_element_type=_f32)
    for h in range(H):
        base = m[:, h * C:(h + 1) * C]
        if C < 128:
            base = jnp.concatenate(
                [base, jnp.zeros((RB, 128 - C), _f32)], axis=1)
        hf_ref[h] = base
    a_ref[...] = m[:, 512:640]


def _tc_matmul(x, wfull, H, C):
    K = x.shape[1]
    grid = (N // RB,)
    return pl.pallas_call(
        functools.partial(_mm_body, H, C),
        grid=grid,
        in_specs=[
            pl.BlockSpec((RB, K), lambda i: (i, 0)),
            pl.BlockSpec((K, 640), lambda i: (0, 0)),
        ],
        out_specs=[
            pl.BlockSpec((H, RB, 128), lambda i: (0, i, 0)),
            pl.BlockSpec((RB, 128), lambda i: (i, 0)),
        ],
        out_shape=[
            jax.ShapeDtypeStruct((H, N, 128), _f32),
            jax.ShapeDtypeStruct((N, 128), _f32),
        ],
    )(x, wfull)


ZB = 40


def _sc_body(H, NP, ECH, NCG, split, hfeat, asrc, adst, src, dst, emask, out,
             atab_s, atab_d, srcv, dstv, mv, gidx, rows, dden, didx, zbuf,
             accS, semg):
    NCHUNK = ((EPAD // (NTILE * NSC)) if split else (E // NTILE)) // ECH
    sc = lax.axis_index("c")
    tid = lax.axis_index("s")
    z16 = jnp.zeros((16,), _f32)
    for gg in range(DROWS // 16):
        didx[pl.ds(gg * 16, 16)] = lax.iota(_i32, 16) + (NPAD + gg * 16)
    for r in range(ZB):
        for v in range(8):
            zbuf[r, pl.ds(v * 16, 16)] = z16

    for p in range(NP):
        if split:
            head = jnp.int32(0)
            obase = sc * OPAD
            ebase = (sc * NTILE + tid) * (EPAD // (NTILE * NSC))
        else:
            head = sc * NP + p
            obase = head * OPAD
            ebase = tid * (E // NTILE)
        pltpu.sync_copy(asrc.at[head], atab_s)
        pltpu.sync_copy(adst.at[head], atab_d)
        for r in range(DROWS):
            for v in range(8):
                dden[r, pl.ds(v * 16, 16)] = z16
        for k in range(ROWS_T // ZB):
            pltpu.sync_copy(zbuf,
                            accS.at[pl.ds(tid * ROWS_T + k * ZB, ZB)])
        pltpu.sync_copy(zbuf.at[pl.ds(0, 8)],
                        accS.at[pl.ds(NPAD + tid * 8, 8)])
        plsc.subcore_barrier()

        def chunk(g, _, head=head, ebase=ebase):
            eb = ebase + g * ECH
            c1 = pltpu.async_copy(src.at[pl.ds(eb, ECH)], srcv, semg)
            c2 = pltpu.async_copy(dst.at[pl.ds(eb, ECH)], dstv, semg)
            c3 = pltpu.async_copy(emask.at[pl.ds(eb, ECH)], mv, semg)
            c1.wait(); c2.wait(); c3.wait()
            for gg in range(ECH // 16):
                sl = pl.ds(gg * 16, 16)
                gidx[sl] = srcv[sl] + head * N
            pltpu.sync_copy(hfeat.at[gidx], rows)
            lane = lax.iota(_i32, 16)
            for gg in range(ECH // 16):
                sl = pl.ds(gg * 16, 16)
                sv = srcv[sl]
                dv = dstv[sl]
                mm = mv[sl]
                av = plsc.load_gather(atab_s, [sv])
                bv = plsc.load_gather(atab_d, [dv])
                s = av + bv
                s = jnp.maximum(s, 0.2 * s)
                exv = jnp.where(mm > 0, jnp.exp(s), 0.0)
                dr = lax.shift_right_logical(dv, 7)
                dc = lax.bitwise_and(dv, 127)
                for jj in range(16):
                    plsc.addupdate_scatter(dden, [dr, dc], exv,
                                           mask=lane == jj)
                    exb = _vgather(exv, jnp.full((16,), jj, _i32))
                    r = gg * 16 + jj
                    for v in range(NCG):
                        fs = pl.ds(v * 16, 16)
                        rows[r, fs] = rows[r, fs] * exb
            pltpu.sync_copy(rows, accS.at[dstv], add=True)
            return 0

        lax.fori_loop(0, NCHUNK, chunk, 0, unroll=False)
        pltpu.sync_copy(dden, accS.at[didx], add=True)
        plsc.subcore_barrier()
        for k in range(ROWS_T // ZB):
            pltpu.sync_copy(accS.at[pl.ds(tid * ROWS_T + k * ZB, ZB)],
                            rows.at[pl.ds(0, ZB)])
            pltpu.sync_copy(rows.at[pl.ds(0, ZB)],
                            out.at[pl.ds(obase + tid * ROWS_T + k * ZB, ZB)])
        pltpu.sync_copy(accS.at[pl.ds(NPAD + tid * 8, 8)],
                        rows.at[pl.ds(0, 8)])
        pltpu.sync_copy(rows.at[pl.ds(0, 8)],
                        out.at[pl.ds(obase + NPAD + tid * 8, 8)])
        plsc.subcore_barrier()


def _sc_conv(hfeat, asrc_f, adst_f, src, dst, emask, H, C, split):
    NP = 1 if split else 2
    ECH = 80
    NCG = (C + 15) // 16
    out_rows = (2 if split else H) * OPAD
    mesh = plsc.VectorSubcoreMesh(core_axis_name="c", subcore_axis_name="s",
                                  num_cores=NSC, num_subcores=NTILE)
    kfn = pl.kernel(
        functools.partial(_sc_body, H, NP, ECH, NCG, split),
        out_type=jax.ShapeDtypeStruct((out_rows, 128), _f32),
        mesh=mesh,
        scratch_types=[
            pltpu.VMEM((N,), _f32),
            pltpu.VMEM((N,), _f32),
            pltpu.VMEM((ECH,), _i32),
            pltpu.VMEM((ECH,), _i32),
            pltpu.VMEM((ECH,), _i32),
            pltpu.VMEM((ECH,), _i32),
            pltpu.VMEM((ECH, 128), _f32),
            pltpu.VMEM((DROWS, 128), _f32),
            pltpu.VMEM((DROWS,), _i32),
            pltpu.VMEM((ZB, 128), _f32),
            pltpu.VMEM_SHARED((OPAD, 128), _f32),
            pltpu.SemaphoreType.DMA,
        ],
        compiler_params=pltpu.CompilerParams(needs_layout_passes=False),
    )
    return kfn(hfeat, asrc_f, adst_f, src, dst, emask)


def _norm_body(H, num_ref, hf_ref, a_ref, dp_ref, b_ref, smat_ref, bbig_ref,
               out_ref):
    a = a_ref[...]
    logit = jnp.dot(a, smat_ref[...], preferred_element_type=_f32)
    logit = jnp.maximum(logit, 0.2 * logit)
    exs = jnp.exp(logit)
    bbig = bbig_ref[...]
    sel = jnp.dot(exs, bbig, preferred_element_type=_f32)
    den = jnp.dot(dp_ref[...], bbig, preferred_element_type=_f32)
    nmain = jnp.concatenate([num_ref[h] for h in range(H)], axis=1)
    hmain = jnp.concatenate([hf_ref[h] for h in range(H)], axis=1)
    val = (nmain + sel * hmain) / (den + sel) + b_ref[...]
    out_ref[...] = jnp.where(val > 0, val, jnp.exp(jnp.minimum(val, 0.0)) - 1.0)


def _tc_norm(num, hf, a, dpack, bias, H):
    smat = jnp.zeros((128, 128), _f32)
    for h in range(H):
        smat = smat.at[h, h].set(1.0).at[H + h, h].set(1.0)
    bbig = jnp.zeros((128, H * 128), _f32)
    for h in range(H):
        bbig = bbig.at[h, h * 128:(h + 1) * 128].set(1.0)
    grid = (N // RB,)
    return pl.pallas_call(
        functools.partial(_norm_body, H),
        grid=grid,
        in_specs=[
            pl.BlockSpec((H, RB, 128), lambda i: (0, i, 0)),
            pl.BlockSpec((H, RB, 128), lambda i: (0, i, 0)),
            pl.BlockSpec((RB, 128), lambda i: (i, 0)),
            pl.BlockSpec((RB, 128), lambda i: (i, 0)),
            pl.BlockSpec((1, H * 128), lambda i: (0, 0)),
            pl.BlockSpec((128, 128), lambda i: (0, 0)),
            pl.BlockSpec((128, H * 128), lambda i: (0, 0)),
        ],
        out_specs=pl.BlockSpec((RB, H * 128), lambda i: (i, 0)),
        out_shape=jax.ShapeDtypeStruct((N, H * 128), _f32),
    )(num, hf, a, dpack, bias.reshape(1, -1), smat, bbig)


def _last_body(num_ref, hf_ref, a_ref, dp_ref, b_ref, s2_ref, out_ref):
    nm = num_ref[0] + num_ref[1]
    logit16 = jnp.dot(a_ref[...], s2_ref[...], preferred_element_type=_f32)
    logit16 = jnp.maximum(logit16, 0.2 * logit16)
    sel = jnp.exp(logit16)
    val = (nm[:, 0:16] + sel * hf_ref[:, 0:16]) / (dp_ref[...] + sel) \
        + b_ref[...]
    mx = jnp.max(val, axis=1, keepdims=True)
    ev = jnp.exp(val - mx)
    out_ref[...] = ev / jnp.sum(ev, axis=1, keepdims=True)


def _tc_last(num, hf, a, dpack16, bias):
    s2 = jnp.zeros((128, 16), _f32).at[0, :].set(1.0).at[1, :].set(1.0)
    grid = (N // RB,)
    return pl.pallas_call(
        _last_body,
        grid=grid,
        in_specs=[
            pl.BlockSpec((2, RB, 128), lambda i: (0, i, 0)),
            pl.BlockSpec((RB, 128), lambda i: (i, 0)),
            pl.BlockSpec((RB, 128), lambda i: (i, 0)),
            pl.BlockSpec((RB, 16), lambda i: (i, 0)),
            pl.BlockSpec((1, 16), lambda i: (0, 0)),
            pl.BlockSpec((128, 16), lambda i: (0, 0)),
        ],
        out_specs=pl.BlockSpec((RB, 16), lambda i: (i, 0)),
        out_shape=jax.ShapeDtypeStruct((N, 16), _f32),
    )(num, hf, a, dpack16, bias.reshape(1, -1), s2)


def _wfull(p, H, C):
    K = p["W"].shape[0]
    wf = jnp.zeros((K, 640), _f32)
    wf = wf.at[:, :H * C].set(p["W"])
    for h in range(H):
        wcol = p["W"][:, h * C:(h + 1) * C]
        wf = wf.at[:, 512 + h].set(wcol @ p["att_src"][h])
        wf = wf.at[:, 512 + H + h].set(wcol @ p["att_dst"][h])
    return wf


def _dpack(num, H):
    den = num.reshape(-1, OPAD, 128)[:, NPAD:NPAD + DROWS, :]
    den = den.reshape(-1, NPAD)[:, :N]
    if den.shape[0] > H:
        den = jnp.sum(den, axis=0, keepdims=True)
    out = jnp.zeros((N, 128), _f32)
    return out.at[:, :H].set(den.T)


def _conv(h_in, src, dst, emask, p, H, C, split):
    wfull = _wfull(p, H, C)
    hf, a = _tc_matmul(h_in, wfull, H, C)
    asrc_f = a[:, 0:H].T
    adst_f = a[:, H:2 * H].T
    num = _sc_conv(hf.reshape(H * N, 128), asrc_f, adst_f, src, dst, emask,
                   H, C, split)
    return hf, a, num


def kernel(x, edge_index, is_reversed, params):
    src = edge_index[0]
    dst = edge_index[1]
    m_st = jnp.where(is_reversed, 0, 1).astype(_i32)
    m_ts = jnp.where(is_reversed, 1, 0).astype(_i32)
    ones = jnp.ones((E,), _i32)

    h = x
    for i in range(2):
        p1, p2 = params["st%d" % i], params["ts%d" % i]
        hf1, a1, num1 = _conv(h, src, dst, m_st, p1, HEADS, CH, False)
        hf2, a2, num2 = _conv(h, src, dst, m_ts, p2, HEADS, CH, False)
        x1 = _tc_norm(num1.reshape(HEADS, OPAD, 128), hf1, a1,
                      _dpack(num1, HEADS), p1["bias"], HEADS)
        x2 = _tc_norm(num2.reshape(HEADS, OPAD, 128), hf2, a2,
                      _dpack(num2, HEADS), p2["bias"], HEADS)
        h = jnp.concatenate([x1, x2], axis=1)

    pl_ = params["last"]
    zpad = jnp.zeros((EPAD - E,), _i32)
    src_p = jnp.concatenate([src, zpad])
    dst_p = jnp.concatenate([dst, zpad])
    ones_p = jnp.concatenate([ones, zpad])
    hfl, al, numl = _conv(h, src_p, dst_p, ones_p, pl_, 1, NUM_CLASSES, True)
    dp16 = jnp.broadcast_to(_dpack(numl, 1)[:, 0:1], (N, 16))
    return _tc_last(numl.reshape(2, OPAD, 128), hfl.reshape(N, 128), al,
                    dp16, pl_["bias"])
